# baseline (speedup 1.0000x reference)
import functools

import numpy as np
import jax
import jax.numpy as jnp
from jax import lax
from jax.experimental import pallas as pl
from jax.experimental.pallas import tpu as pltpu
from jax.experimental.pallas import tpu_sc as plsc

_D = 768
_PE_LEN = 2048
_BATCH = 4
_SCALE = float(np.sqrt(float(_D)))

_NC = 2
_NS = 16
_NW = _NC * _NS
_B = _BATCH * _PE_LEN
_T_PER_W = _PE_LEN // _NW
_ROWS_PER_W = _BATCH * _T_PER_W
_SUB = 32
_NSUB = _ROWS_PER_W // _SUB
_HPB = _T_PER_W // _SUB
_HALF = _SUB // 2
_LANES = 16
_VPR = _D // _LANES
_NBUF = 4


_GRP = _VPR // 2
_DW = _D // 2


def _pe_packed() -> np.ndarray:
    half = _D / 2
    positions = np.arange(_PE_LEN)[:, np.newaxis]
    depths = np.arange(half)[np.newaxis, :] / half
    angle_rates = 1.0 / (10000.0 ** depths)
    angle_rads = positions * angle_rates
    pe = np.concatenate([np.sin(angle_rads), np.cos(angle_rads)], axis=-1)
    pe = pe.astype(np.float32)
    bits = pe.view(np.uint32)
    bf16 = ((bits + 0x7FFF + ((bits >> 16) & 1)) >> 16).astype(np.uint32)
    bf16 = bf16.reshape(_PE_LEN, _GRP, 2, _LANES)
    packed = bf16[:, :, 0, :] | (bf16[:, :, 1, :] << 16)
    return packed.reshape(_PE_LEN, _DW).astype(np.int32)


_PE_CONST = _pe_packed()

_mesh = plsc.VectorSubcoreMesh(core_axis_name="c", subcore_axis_name="s")


@functools.partial(
    pl.kernel,
    out_type=jax.ShapeDtypeStruct((_B, _D), jnp.float32),
    mesh=_mesh,
    scratch_types=[
        pltpu.VMEM((_ROWS_PER_W,), jnp.int32),
        pltpu.VMEM((_T_PER_W, _DW), jnp.int32),
        [pltpu.VMEM((_SUB, _D), jnp.float32) for _ in range(_NBUF)],
        [pltpu.SemaphoreType.DMA for _ in range(_NBUF)],
        [pltpu.SemaphoreType.DMA for _ in range(_NBUF)],
        pltpu.SemaphoreType.DMA,
    ],
)
def _sc_embed(x_hbm, table_hbm, pe_hbm, out_hbm,
              idx_v, pe_pk, em_v, sem_g, sem_o, sem_pe):
    c = lax.axis_index("c")
    s = lax.axis_index("s")
    wid = s * _NC + c
    t0 = wid * _T_PER_W

    pltpu.async_copy(pe_hbm.at[pl.ds(t0, _T_PER_W)], pe_pk, sem_pe)

    for bt in range(_BATCH):
        pltpu.sync_copy(
            x_hbm.at[pl.ds(bt * _PE_LEN + t0, _T_PER_W)],
            idx_v.at[pl.ds(bt * _T_PER_W, _T_PER_W)],
        )

    def out_base(k):
        bt, h = k // _HPB, k % _HPB
        return bt * _PE_LEN + t0 + h * _SUB

    def start_gather(k):
        b = k % _NBUF
        pltpu.async_copy(
            table_hbm.at[idx_v.at[pl.ds(k * _SUB, _SUB)]], em_v[b], sem_g[b]
        )

    for k in range(min(_NBUF - 1, _NSUB)):
        start_gather(k)

    pltpu.make_async_copy(
        pe_hbm.at[pl.ds(t0, _T_PER_W)], pe_pk, sem_pe
    ).wait()

    for k in range(_NSUB):
        b = k % _NBUF
        if k + _NBUF - 1 < _NSUB:
            kk = k + _NBUF - 1
            if kk >= _NBUF:
                for hh in range(2):
                    pltpu.make_async_copy(
                        em_v[kk % _NBUF].at[pl.ds(hh * _HALF, _HALF)],
                        out_hbm.at[
                            pl.ds(out_base(kk - _NBUF) + hh * _HALF, _HALF)
                        ],
                        sem_o[kk % _NBUF],
                    ).wait()
            start_gather(kk)
        pltpu.make_async_copy(
            table_hbm.at[idx_v.at[pl.ds(k * _SUB, _SUB)]], em_v[b], sem_g[b]
        ).wait()
        ph = (k % _HPB) * _SUB

        for hh in range(2):
            @plsc.parallel_loop(hh * _HALF, (hh + 1) * _HALF)
            def _fma_row(r):
                erow = em_v[b].at[r]
                prow = pe_pk.at[ph + r]
                for g in range(_GRP):
                    bits = prow[pl.ds(g * _LANES, _LANES)]
                    pa = lax.bitcast_convert_type(bits << 16, jnp.float32)
                    pb = lax.bitcast_convert_type(
                        bits & jnp.int32(-65536), jnp.float32
                    )
                    sla = pl.ds(g * 2 * _LANES, _LANES)
                    slb = pl.ds((g * 2 + 1) * _LANES, _LANES)
                    erow[sla] = erow[sla] * _SCALE + pa
                    erow[slb] = erow[slb] * _SCALE + pb

            pltpu.async_copy(
                em_v[b].at[pl.ds(hh * _HALF, _HALF)],
                out_hbm.at[pl.ds(out_base(k) + hh * _HALF, _HALF)],
                sem_o[b],
            )

    for k in range(_NSUB - _NBUF, _NSUB):
        b = k % _NBUF
        for hh in range(2):
            pltpu.make_async_copy(
                em_v[b].at[pl.ds(hh * _HALF, _HALF)],
                out_hbm.at[pl.ds(out_base(k) + hh * _HALF, _HALF)],
                sem_o[b],
            ).wait()


def kernel(x, table):
    pe = jnp.asarray(_PE_CONST)
    xf = x.reshape(-1).astype(jnp.int32)
    out = _sc_embed(xf, table, pe)
    return out.reshape(_BATCH, _PE_LEN, _D)


# speedup vs baseline: 1.0919x; 1.0919x over previous
import functools

import numpy as np
import jax
import jax.numpy as jnp
from jax import lax
from jax.experimental import pallas as pl
from jax.experimental.pallas import tpu as pltpu
from jax.experimental.pallas import tpu_sc as plsc

_D = 768
_PE_LEN = 2048
_BATCH = 4
_SCALE = float(np.sqrt(float(_D)))

_NC = 2
_NS = 16
_NW = _NC * _NS
_B = _BATCH * _PE_LEN
_T_PER_W = _PE_LEN // _NW
_ROWS_PER_W = _BATCH * _T_PER_W
_SUB = 32
_NSUB = _ROWS_PER_W // _SUB
_HPB = _T_PER_W // _SUB
_HALF = _SUB // 2
_LANES = 16
_VPR = _D // _LANES
_NBUF = 4


_GRP = _VPR // 2
_DW = _D // 2


def _pe_packed() -> np.ndarray:
    half = _D / 2
    positions = np.arange(_PE_LEN)[:, np.newaxis]
    depths = np.arange(half)[np.newaxis, :] / half
    angle_rates = 1.0 / (10000.0 ** depths)
    angle_rads = positions * angle_rates
    pe = np.concatenate([np.sin(angle_rads), np.cos(angle_rads)], axis=-1)
    pe = pe.astype(np.float32)
    bits = pe.view(np.uint32)
    bf16 = ((bits + 0x7FFF + ((bits >> 16) & 1)) >> 16).astype(np.uint32)
    bf16 = bf16.reshape(_PE_LEN, _GRP, 2, _LANES)
    packed = bf16[:, :, 0, :] | (bf16[:, :, 1, :] << 16)
    return packed.reshape(_PE_LEN, _DW).astype(np.int32)


_PE_CONST = _pe_packed()

_mesh = plsc.VectorSubcoreMesh(core_axis_name="c", subcore_axis_name="s")


@functools.partial(
    pl.kernel,
    out_type=jax.ShapeDtypeStruct((_B, _D), jnp.float32),
    mesh=_mesh,
    scratch_types=[
        pltpu.VMEM((_ROWS_PER_W,), jnp.int32),
        pltpu.VMEM((_T_PER_W, _DW), jnp.int32),
        [pltpu.VMEM((_SUB, _D), jnp.float32) for _ in range(_NBUF)],
        [pltpu.SemaphoreType.DMA for _ in range(_NBUF)],
        [pltpu.SemaphoreType.DMA for _ in range(_NBUF)],
        pltpu.SemaphoreType.DMA,
    ],
)
def _sc_embed(x_hbm, table_hbm, pe_hbm, out_hbm,
              idx_v, pe_pk, em_v, sem_g, sem_o, sem_pe):
    c = lax.axis_index("c")
    s = lax.axis_index("s")
    wid = s * _NC + c
    t0 = wid * _T_PER_W

    pltpu.async_copy(pe_hbm.at[pl.ds(t0, _T_PER_W)], pe_pk, sem_pe)

    for bt in range(_BATCH):
        pltpu.sync_copy(
            x_hbm.at[pl.ds(bt * _PE_LEN + t0, _T_PER_W)],
            idx_v.at[pl.ds(bt * _T_PER_W, _T_PER_W)],
        )

    def out_base(k):
        bt, h = k // _HPB, k % _HPB
        return bt * _PE_LEN + t0 + h * _SUB

    def start_gather(k):
        b = k % _NBUF
        pltpu.async_copy(
            table_hbm.at[idx_v.at[pl.ds(k * _SUB, _SUB)]], em_v[b], sem_g[b]
        )

    for k in range(min(_NBUF - 1, _NSUB)):
        start_gather(k)

    pltpu.make_async_copy(
        pe_hbm.at[pl.ds(t0, _T_PER_W)], pe_pk, sem_pe
    ).wait()

    for k in range(_NSUB):
        b = k % _NBUF
        if k + _NBUF - 1 < _NSUB:
            kk = k + _NBUF - 1
            if kk >= _NBUF:
                pltpu.make_async_copy(
                    em_v[kk % _NBUF],
                    out_hbm.at[pl.ds(out_base(kk - _NBUF), _SUB)],
                    sem_o[kk % _NBUF],
                ).wait()
            start_gather(kk)
        pltpu.make_async_copy(
            table_hbm.at[idx_v.at[pl.ds(k * _SUB, _SUB)]], em_v[b], sem_g[b]
        ).wait()
        ph = (k % _HPB) * _SUB

        @plsc.parallel_loop(0, _SUB)
        def _fma_row(r):
            erow = em_v[b].at[r]
            prow = pe_pk.at[ph + r]
            for g in range(_GRP):
                bits = prow[pl.ds(g * _LANES, _LANES)]
                pa = lax.bitcast_convert_type(bits << 16, jnp.float32)
                pb = lax.bitcast_convert_type(
                    bits & jnp.int32(-65536), jnp.float32
                )
                sla = pl.ds(g * 2 * _LANES, _LANES)
                slb = pl.ds((g * 2 + 1) * _LANES, _LANES)
                erow[sla] = erow[sla] * _SCALE + pa
                erow[slb] = erow[slb] * _SCALE + pb

        pltpu.async_copy(
            em_v[b], out_hbm.at[pl.ds(out_base(k), _SUB)], sem_o[b]
        )

    for k in range(_NSUB - _NBUF, _NSUB):
        b = k % _NBUF
        pltpu.make_async_copy(
            em_v[b], out_hbm.at[pl.ds(out_base(k), _SUB)], sem_o[b]
        ).wait()


def kernel(x, table):
    pe = jnp.asarray(_PE_CONST)
    xf = x.reshape(-1).astype(jnp.int32)
    out = _sc_embed(xf, table, pe)
    return out.reshape(_BATCH, _PE_LEN, _D)


# speedup vs baseline: 1.0965x; 1.0041x over previous
import functools

import numpy as np
import jax
import jax.numpy as jnp
from jax import lax
from jax.experimental import pallas as pl
from jax.experimental.pallas import tpu as pltpu
from jax.experimental.pallas import tpu_sc as plsc

_D = 768
_PE_LEN = 2048
_BATCH = 4
_SCALE = float(np.sqrt(float(_D)))

_NC = 2
_NS = 16
_NW = _NC * _NS
_B = _BATCH * _PE_LEN
_T_PER_W = _PE_LEN // _NW
_ROWS_PER_W = _BATCH * _T_PER_W
_SUB = 32
_NSUB = _ROWS_PER_W // _SUB
_HPB = _T_PER_W // _SUB
_LANES = 16
_VPR = _D // _LANES
_NBUF = 4


_GRP = _VPR // 2
_DW = _D // 2


def _pe_packed() -> np.ndarray:
    half = _D / 2
    positions = np.arange(_PE_LEN)[:, np.newaxis]
    depths = np.arange(half)[np.newaxis, :] / half
    angle_rates = 1.0 / (10000.0 ** depths)
    angle_rads = positions * angle_rates
    pe = np.concatenate([np.sin(angle_rads), np.cos(angle_rads)], axis=-1)
    pe = pe.astype(np.float32)
    bits = pe.view(np.uint32)
    bf16 = ((bits + 0x7FFF + ((bits >> 16) & 1)) >> 16).astype(np.uint32)
    bf16 = bf16.reshape(_PE_LEN, _GRP, 2, _LANES)
    packed = bf16[:, :, 0, :] | (bf16[:, :, 1, :] << 16)
    return packed.reshape(_PE_LEN, _DW).astype(np.int32)


_PE_CONST = _pe_packed()

_mesh = plsc.VectorSubcoreMesh(core_axis_name="c", subcore_axis_name="s")


@functools.partial(
    pl.kernel,
    out_type=jax.ShapeDtypeStruct((_B, _D), jnp.float32),
    mesh=_mesh,
    scratch_types=[
        pltpu.VMEM((_ROWS_PER_W,), jnp.int32),
        pltpu.VMEM((_T_PER_W, _DW), jnp.int32),
        [pltpu.VMEM((_SUB, _D), jnp.float32) for _ in range(_NBUF)],
        [pltpu.SemaphoreType.DMA for _ in range(_NBUF)],
        [pltpu.SemaphoreType.DMA for _ in range(_NBUF)],
        pltpu.SemaphoreType.DMA,
    ],
)
def _sc_embed(x_hbm, table_hbm, pe_hbm, out_hbm,
              idx_v, pe_pk, em_v, sem_g, sem_o, sem_pe):
    c = lax.axis_index("c")
    s = lax.axis_index("s")
    wid = s * _NC + c
    t0 = wid * _T_PER_W

    pltpu.async_copy(pe_hbm.at[pl.ds(t0, _T_PER_W)], pe_pk, sem_pe)

    for bt in range(_BATCH):
        pltpu.sync_copy(
            x_hbm.at[pl.ds(bt * _PE_LEN + t0, _T_PER_W)],
            idx_v.at[pl.ds(bt * _T_PER_W, _T_PER_W)],
        )

    def out_base(k):
        bt, h = k // _HPB, k % _HPB
        return bt * _PE_LEN + t0 + h * _SUB

    def start_gather(k):
        b = k % _NBUF
        pltpu.async_copy(
            table_hbm.at[idx_v.at[pl.ds(k * _SUB, _SUB)]], em_v[b], sem_g[b]
        )

    for k in range(min(_NBUF - 1, _NSUB)):
        start_gather(k)

    pltpu.make_async_copy(
        pe_hbm.at[pl.ds(t0, _T_PER_W)], pe_pk, sem_pe
    ).wait()

    for k in range(_NSUB):
        b = k % _NBUF
        if k + _NBUF - 1 < _NSUB:
            kk = k + _NBUF - 1
            if kk >= _NBUF:
                pltpu.make_async_copy(
                    em_v[kk % _NBUF],
                    out_hbm.at[pl.ds(out_base(kk - _NBUF), _SUB)],
                    sem_o[kk % _NBUF],
                ).wait()
            start_gather(kk)
        pltpu.make_async_copy(
            table_hbm.at[idx_v.at[pl.ds(k * _SUB, _SUB)]], em_v[b], sem_g[b]
        ).wait()
        ph = (k % _HPB) * _SUB

        @plsc.parallel_loop(0, _SUB)
        def _fma_row(r):
            erow = em_v[b].at[r]
            prow = pe_pk.at[ph + r]
            for g in range(_GRP):
                bits = prow[pl.ds(g * _LANES, _LANES)]
                pa = lax.bitcast_convert_type(bits << 16, jnp.float32)
                pb = lax.bitcast_convert_type(
                    bits & jnp.int32(-65536), jnp.float32
                )
                sla = pl.ds(g * 2 * _LANES, _LANES)
                slb = pl.ds((g * 2 + 1) * _LANES, _LANES)
                erow[sla] = erow[sla] * _SCALE + pa
                erow[slb] = erow[slb] * _SCALE + pb

        pltpu.async_copy(
            em_v[b], out_hbm.at[pl.ds(out_base(k), _SUB)], sem_o[b]
        )

    for k in range(_NSUB - _NBUF, _NSUB):
        b = k % _NBUF
        pltpu.make_async_copy(
            em_v[b], out_hbm.at[pl.ds(out_base(k), _SUB)], sem_o[b]
        ).wait()


def kernel(x, table):
    pe = jnp.asarray(_PE_CONST)
    xf = x.reshape(-1).astype(jnp.int32)
    out = _sc_embed(xf, table, pe)
    return out.reshape(_BATCH, _PE_LEN, _D)
